# baseline (speedup 1.0000x reference)
import functools

import jax
import jax.numpy as jnp
from jax import lax
from jax.experimental import pallas as pl
from jax.experimental.pallas import tpu as pltpu
from jax.experimental.pallas import tpu_sc as plsc

BATCH = 4096
SEQ_LEN = 200
EMBED_DIM = 64

NUM_CORES = 2
NUM_SUBCORES = 16
NUM_WORKERS = NUM_CORES * NUM_SUBCORES

TOTAL = BATCH * SEQ_LEN
PER_WORKER = TOTAL // NUM_WORKERS
CHUNK = 128
NCHUNK = PER_WORKER // CHUNK


def _emb_kernel(idx_hbm, table_hbm, out_hbm, idx_v, rows_v, gsem):
    wid = lax.axis_index("s") * NUM_CORES + lax.axis_index("c")
    pltpu.sync_copy(idx_hbm.at[wid], idx_v)

    def body(j, carry):
        pltpu.async_copy(table_hbm.at[idx_v.at[j]], rows_v, gsem).wait()
        pltpu.sync_copy(rows_v, out_hbm.at[wid, j])
        return carry

    lax.fori_loop(0, NCHUNK, body, 0)


def kernel(x, weight):
    idx = x.astype(jnp.int32).reshape(NUM_WORKERS, NCHUNK, CHUNK)
    mesh = plsc.VectorSubcoreMesh(core_axis_name="c", subcore_axis_name="s")

    emb = functools.partial(
        pl.kernel,
        mesh=mesh,
        out_type=jax.ShapeDtypeStruct(
            (NUM_WORKERS, NCHUNK, CHUNK, EMBED_DIM), jnp.float32
        ),
        scratch_types=[
            pltpu.VMEM((NCHUNK, CHUNK), jnp.int32),
            pltpu.VMEM((CHUNK, EMBED_DIM), jnp.float32),
            pltpu.SemaphoreType.DMA,
        ],
        compiler_params=pltpu.CompilerParams(use_tc_tiling_on_sc=False),
    )(_emb_kernel)

    out = emb(idx, weight)
    return out.reshape(BATCH, SEQ_LEN, EMBED_DIM)


# speedup vs baseline: 1.1168x; 1.1168x over previous
import functools

import jax
import jax.numpy as jnp
from jax import lax
from jax.experimental import pallas as pl
from jax.experimental.pallas import tpu as pltpu
from jax.experimental.pallas import tpu_sc as plsc

BATCH = 4096
SEQ_LEN = 200
EMBED_DIM = 64

NUM_CORES = 2
NUM_SUBCORES = 16
NUM_WORKERS = NUM_CORES * NUM_SUBCORES

TOTAL = BATCH * SEQ_LEN
PER_WORKER = TOTAL // NUM_WORKERS
CHUNK = 128
NCHUNK = PER_WORKER // CHUNK
NBUF = 4


def _emb_kernel(idx_hbm, table_hbm, out_hbm, idx_v, rows_v, gsem, osem):
    wid = lax.axis_index("s") * NUM_CORES + lax.axis_index("c")
    pltpu.sync_copy(idx_hbm.at[wid], idx_v)

    def gather(j, b):
        pltpu.async_copy(table_hbm.at[idx_v.at[j]], rows_v.at[b], gsem.at[b])

    def gather_wait(j, b):
        pltpu.make_async_copy(
            table_hbm.at[idx_v.at[j]], rows_v.at[b], gsem.at[b]
        ).wait()

    def writeout(j, b):
        pltpu.async_copy(rows_v.at[b], out_hbm.at[wid, j], osem.at[b])

    def writeout_wait(j, b):
        pltpu.make_async_copy(
            rows_v.at[b], out_hbm.at[wid, j], osem.at[b]
        ).wait()

    for b in range(NBUF):
        gather(b, b)

    def body(jj, carry):
        for b in range(NBUF):
            j = jj * NBUF + b
            gather_wait(j, b)
            writeout(j, b)

            @pl.when(j + NBUF < NCHUNK)
            def _():
                writeout_wait(j, b)
                gather(j + NBUF, b)

        return carry

    lax.fori_loop(0, NCHUNK // NBUF, body, 0)

    for b in range(NBUF):
        writeout_wait(NCHUNK - NBUF + b, b)


def kernel(x, weight):
    idx = x.astype(jnp.int32).reshape(NUM_WORKERS, NCHUNK, CHUNK)
    mesh = plsc.VectorSubcoreMesh(core_axis_name="c", subcore_axis_name="s")

    emb = functools.partial(
        pl.kernel,
        mesh=mesh,
        out_type=jax.ShapeDtypeStruct(
            (NUM_WORKERS, NCHUNK, CHUNK, EMBED_DIM), jnp.float32
        ),
        scratch_types=[
            pltpu.VMEM((NCHUNK, CHUNK), jnp.int32),
            pltpu.VMEM((NBUF, CHUNK, EMBED_DIM), jnp.float32),
            pltpu.SemaphoreType.DMA((NBUF,)),
            pltpu.SemaphoreType.DMA((NBUF,)),
        ],
        compiler_params=pltpu.CompilerParams(use_tc_tiling_on_sc=False),
    )(_emb_kernel)

    out = emb(idx, weight)
    return out.reshape(BATCH, SEQ_LEN, EMBED_DIM)


# speedup vs baseline: 1.9115x; 1.7115x over previous
import functools

import jax
import jax.numpy as jnp
from jax import lax
from jax.experimental import pallas as pl
from jax.experimental.pallas import tpu as pltpu
from jax.experimental.pallas import tpu_sc as plsc

BATCH = 4096
SEQ_LEN = 200
EMBED_DIM = 64
PAD_DIM = 128
VOCAB = 1000000

NUM_CORES = 2
NUM_SUBCORES = 16
NUM_WORKERS = NUM_CORES * NUM_SUBCORES

ROWS_PER_W = BATCH // NUM_WORKERS
HALF_A = 104
HALF_B = SEQ_LEN - HALF_A
NBUF = 2


def _emb_kernel(idx_hbm, table_hbm, out_hbm, idx_v, rows_v, gsem, osem):
    wid = lax.axis_index("s") * NUM_CORES + lax.axis_index("c")
    pltpu.sync_copy(idx_hbm.at[wid], idx_v)

    def chunks(il, b):
        base = il * SEQ_LEN
        yield idx_v.at[pl.ds(base, HALF_A)], rows_v.at[b, pl.ds(0, HALF_A)]
        yield (
            idx_v.at[pl.ds(base + HALF_A, HALF_B)],
            rows_v.at[b, pl.ds(HALF_A, HALF_B)],
        )

    def gather(il, b):
        for isl, rsl in chunks(il, b):
            pltpu.async_copy(table_hbm.at[isl], rsl, gsem.at[b])

    def gather_wait(il, b):
        for isl, rsl in chunks(il, b):
            pltpu.make_async_copy(table_hbm.at[isl], rsl, gsem.at[b]).wait()

    def writeout(il, b):
        pltpu.async_copy(
            rows_v.at[b], out_hbm.at[wid * ROWS_PER_W + il], osem.at[b]
        )

    def writeout_wait(il, b):
        pltpu.make_async_copy(
            rows_v.at[b], out_hbm.at[wid * ROWS_PER_W + il], osem.at[b]
        ).wait()

    for b in range(NBUF):
        gather(b, b)

    def body(jj, carry):
        for b in range(NBUF):
            il = jj * NBUF + b
            gather_wait(il, b)
            writeout(il, b)

            @pl.when(il + NBUF < ROWS_PER_W)
            def _():
                writeout_wait(il, b)
                gather(il + NBUF, b)

        return carry

    lax.fori_loop(0, ROWS_PER_W // NBUF, body, 0)

    for b in range(NBUF):
        writeout_wait(ROWS_PER_W - NBUF + b, b)


def kernel(x, weight):
    idx = x.astype(jnp.int32).reshape(NUM_WORKERS, ROWS_PER_W * SEQ_LEN)
    proj = jnp.concatenate(
        [jnp.eye(EMBED_DIM, dtype=jnp.float32),
         jnp.zeros((EMBED_DIM, PAD_DIM - EMBED_DIM), jnp.float32)],
        axis=1,
    )
    table = weight @ proj
    mesh = plsc.VectorSubcoreMesh(core_axis_name="c", subcore_axis_name="s")

    emb = functools.partial(
        pl.kernel,
        mesh=mesh,
        out_type=jax.ShapeDtypeStruct((BATCH, SEQ_LEN, PAD_DIM), jnp.float32),
        scratch_types=[
            pltpu.VMEM((ROWS_PER_W * SEQ_LEN,), jnp.int32),
            pltpu.VMEM((NBUF, SEQ_LEN, PAD_DIM), jnp.float32),
            pltpu.SemaphoreType.DMA((NBUF,)),
            pltpu.SemaphoreType.DMA((NBUF,)),
        ],
    )(_emb_kernel)

    return emb(idx, table)[..., :EMBED_DIM]
